# baseline (speedup 1.0000x reference)
import math

import jax
import jax.numpy as jnp
from jax.experimental import pallas as pl
from jax.experimental.pallas import tpu as pltpu

B, Q, N, D = 4, 300, 16384, 256
QP = 304
NHEAD = 8
DH = D // NHEAD
ALIGN_TOPK = 96
CROSS_TOPK = 2048
NEG = -1e30
TN = 2048
NT = N // TN
TR = 64
SCALE = 1.0 / math.sqrt(DH)


def _f32_to_key(x):
    u = jax.lax.bitcast_convert_type(x, jnp.int32)
    return jnp.where(u >= 0, u, u ^ jnp.int32(0x7FFFFFFF))


def _key_to_f32(k):
    u = jnp.where(k >= 0, k, k ^ jnp.int32(0x7FFFFFFF))
    return jax.lax.bitcast_convert_type(u, jnp.float32)


def _kth_largest_key(keys, k):
    def count_ge(t):
        return jnp.sum((keys >= t).astype(jnp.int32), axis=-1, keepdims=True)

    t = jnp.where(count_ge(jnp.int32(0)) >= k,
                  jnp.int32(0), jnp.int32(-2147483648))
    for b in range(30, -1, -1):
        trial = t | jnp.int32(1 << b)
        t = jnp.where(count_ge(trial) >= k, trial, t)
    return t


def _layer_norm(x, w, b, eps=1e-5):
    mu = jnp.mean(x, axis=-1, keepdims=True)
    xc = x - mu
    var = jnp.mean(xc * xc, axis=-1, keepdims=True)
    return xc * jax.lax.rsqrt(var + eps) * w + b


def _gelu(x):
    return 0.5 * x * (1.0 + jax.lax.erf(x * (1.0 / math.sqrt(2.0))))


def _normalize(x, eps=1e-6):
    n = jnp.sqrt(jnp.sum(x * x, axis=-1, keepdims=True))
    return x / jnp.maximum(n, eps)


def _bdot(a, b):
    return jax.lax.dot(a.astype(jnp.bfloat16), b.astype(jnp.bfloat16),
                       preferred_element_type=jnp.float32)


def _softmax(x):
    m = jnp.max(x, axis=-1, keepdims=True)
    e = jnp.exp(x - m)
    return e / jnp.sum(e, axis=-1, keepdims=True)


def _k0_body(q_ref, awq_ref, qn_ref):
    qn_ref[0] = _normalize(q_ref[0] @ awq_ref[...].T)


def _k1_body(qn_ref, mem_ref, awm_ref, cwm_ref, keys_ref, cm_ref):
    mem = mem_ref[0]
    mn = _normalize(mem @ awm_ref[...].T)
    cm_ref[0] = _normalize(mem @ cwm_ref[...].T)
    keys_ref[0] = _f32_to_key(qn_ref[0] @ mn.T)


def _k2a_body(keys_ref, t_ref, m_ref, den_ref):
    keys = keys_ref[...]
    t = _kth_largest_key(keys, ALIGN_TOPK)
    m_key = jnp.max(keys, axis=-1, keepdims=True)
    m = _key_to_f32(m_key)
    sim = _key_to_f32(keys)
    den = jnp.sum(jnp.where(keys >= t, jnp.exp(sim - m), 0.0),
                  axis=-1, keepdims=True)
    t_ref[...] = t
    m_ref[...] = m
    den_ref[...] = den


def _k2b_body(keys_ref, t_ref, m_ref, den_ref, mem_ref, q_ref,
              gw1_ref, gb1_ref, gw2_ref, gb2_ref, out_ref, acc_ref):
    nt = pl.program_id(1)
    keys = keys_ref[0]
    sim = _key_to_f32(keys)
    w = jnp.where(keys >= t_ref[0], jnp.exp(sim - m_ref[0]), 0.0) / den_ref[0]
    part = jax.lax.dot(w.astype(jnp.bfloat16),
                       mem_ref[0].astype(jnp.bfloat16),
                       preferred_element_type=jnp.float32)

    @pl.when(nt == 0)
    def _():
        acc_ref[...] = jnp.zeros_like(acc_ref)

    acc_ref[...] += part

    @pl.when(nt == NT - 1)
    def _():
        aligned = acc_ref[...]
        qb = q_ref[0]
        h = (jnp.concatenate([qb, aligned], axis=-1) @ gw1_ref[...].T
             + gb1_ref[...])
        gate = jax.nn.sigmoid(_gelu(h) @ gw2_ref[...].T + gb2_ref[...])
        out_ref[0] = qb + gate * aligned


def _k3_body(q_ref, qpos_ref, wi_ref, bi_ref, wo_ref, bo_ref,
             ln1w_ref, ln1b_ref, cwq_ref, qln_ref, cq_ref):
    qb = q_ref[0]
    qk = qb + qpos_ref[0]
    wi = wi_ref[...]
    bi = bi_ref[...]
    qh = qk @ wi[:D].T + bi[:D]
    kh = qk @ wi[D:2 * D].T + bi[D:2 * D]
    vh = qb @ wi[2 * D:].T + bi[2 * D:]
    colmask = jnp.where(
        jax.lax.broadcasted_iota(jnp.int32, (QP, QP), 1) < Q, 0.0, NEG)
    outs = []
    for h in range(NHEAD):
        s = slice(h * DH, (h + 1) * DH)
        sc = (qh[:, s] @ kh[:, s].T) * SCALE + colmask
        outs.append(_softmax(sc) @ vh[:, s])
    o = jnp.concatenate(outs, axis=-1) @ wo_ref[...].T + bo_ref[...]
    qln = _layer_norm(qb + o, ln1w_ref[...], ln1b_ref[...])
    qln_ref[0] = qln
    cq_ref[0] = _normalize(qln @ cwq_ref[...].T)


def _k4a_body(cq_ref, cm_ref, imp_ref):
    sim2 = cq_ref[0] @ cm_ref[0].T
    rowmask = jnp.where(
        jax.lax.broadcasted_iota(jnp.int32, (QP, TN), 0) < Q, 0.0, NEG)
    imp_ref[0] = jnp.max(sim2 + rowmask, axis=0, keepdims=True)


def _k4b_body(imp_ref, bias_ref):
    keys = _f32_to_key(imp_ref[0])
    t = _kth_largest_key(keys, CROSS_TOPK)
    bias_ref[0] = jnp.where(keys >= t, 0.0, NEG)


def _k5_body(qln_ref, qpos_ref, mem_ref, bias_ref,
             wi_ref, bi_ref, wo_ref, bo_ref,
             ln2w_ref, ln2b_ref, fw1_ref, fb1_ref, fw2_ref, fb2_ref,
             ln3w_ref, ln3b_ref, out_ref, acc_ref, m_ref, l_ref):
    nt = pl.program_id(1)
    qln = qln_ref[0]
    qi = qln + qpos_ref[0]
    mem = mem_ref[0]
    bias = bias_ref[0]
    wi = wi_ref[...]
    bi = bi_ref[...]

    @pl.when(nt == 0)
    def _():
        acc_ref[...] = jnp.zeros_like(acc_ref)
        m_ref[...] = jnp.full_like(m_ref, NEG)
        l_ref[...] = jnp.zeros_like(l_ref)

    for h in range(NHEAD):
        s = slice(h * DH, (h + 1) * DH)
        qh = _bdot(qi, wi[:D][s].T) + bi[:D][s]
        kh = _bdot(mem, wi[D:2 * D][s].T) + bi[D:2 * D][s]
        vh = _bdot(mem, wi[2 * D:][s].T) + bi[2 * D:][s]
        sc = _bdot(qh, kh.T) * SCALE + bias
        m_old = m_ref[:, h:h + 1]
        m_new = jnp.maximum(m_old, jnp.max(sc, axis=-1, keepdims=True))
        p = jnp.exp(sc - m_new)
        corr = jnp.exp(m_old - m_new)
        l_ref[:, h:h + 1] = l_ref[:, h:h + 1] * corr + jnp.sum(
            p, axis=-1, keepdims=True)
        acc_ref[:, s] = acc_ref[:, s] * corr + _bdot(p, vh)
        m_ref[:, h:h + 1] = m_new

    @pl.when(nt == NT - 1)
    def _():
        outs = []
        for h in range(NHEAD):
            s = slice(h * DH, (h + 1) * DH)
            outs.append(acc_ref[:, s] / l_ref[:, h:h + 1])
        o = jnp.concatenate(outs, axis=-1) @ wo_ref[...].T + bo_ref[...]
        x = _layer_norm(qln + o, ln2w_ref[...], ln2b_ref[...])
        ff = (_gelu(x @ fw1_ref[...].T + fb1_ref[...]) @ fw2_ref[...].T
              + fb2_ref[...])
        out_ref[0] = _layer_norm(x + ff, ln3w_ref[...], ln3b_ref[...])


def _full(arr_shape):
    return pl.BlockSpec(arr_shape, lambda *_: (0,) * len(arr_shape))


def _bq(shape):
    return pl.BlockSpec((1,) + shape, lambda b, *_: (b, 0, 0))


@jax.jit
def kernel(q, q_pos, memory, params):
    p = params
    f32 = jnp.float32
    qpad = jnp.pad(q, ((0, 0), (0, QP - Q), (0, 0)))
    qpos = jnp.pad(q_pos, ((0, 0), (0, QP - Q), (0, 0)))

    qn = pl.pallas_call(
        _k0_body,
        grid=(B,),
        in_specs=[_bq((QP, D)), _full((D, D))],
        out_specs=_bq((QP, D)),
        out_shape=jax.ShapeDtypeStruct((B, QP, D), f32),
    )(qpad, p['align_wq'])

    keys, cm = pl.pallas_call(
        _k1_body,
        grid=(B, NT),
        in_specs=[_bq((QP, D)),
                  pl.BlockSpec((1, TN, D), lambda b, n: (b, n, 0)),
                  _full((D, D)), _full((D, D))],
        out_specs=[pl.BlockSpec((1, QP, TN), lambda b, n: (b, 0, n)),
                   pl.BlockSpec((1, TN, D), lambda b, n: (b, n, 0))],
        out_shape=[jax.ShapeDtypeStruct((B, QP, N), jnp.int32),
                   jax.ShapeDtypeStruct((B, N, D), f32)],
    )(qn, memory, p['align_wm'], p['cross_wm'])

    keys2d = keys.reshape(B * QP, N)
    RT = (B * QP) // TR
    tt, mm, den = pl.pallas_call(
        _k2a_body,
        grid=(RT,),
        in_specs=[pl.BlockSpec((TR, N), lambda r: (r, 0))],
        out_specs=[pl.BlockSpec((TR, 1), lambda r: (r, 0))] * 3,
        out_shape=[jax.ShapeDtypeStruct((B * QP, 1), jnp.int32),
                   jax.ShapeDtypeStruct((B * QP, 1), f32),
                   jax.ShapeDtypeStruct((B * QP, 1), f32)],
    )(keys2d)
    tt = tt.reshape(B, QP, 1)
    mm = mm.reshape(B, QP, 1)
    den = den.reshape(B, QP, 1)

    q1 = pl.pallas_call(
        _k2b_body,
        grid=(B, NT),
        in_specs=[pl.BlockSpec((1, QP, TN), lambda b, n: (b, 0, n)),
                  _bq((QP, 1)), _bq((QP, 1)), _bq((QP, 1)),
                  pl.BlockSpec((1, TN, D), lambda b, n: (b, n, 0)),
                  _bq((QP, D)),
                  _full((D, 2 * D)), _full((D,)), _full((D, D)), _full((D,))],
        out_specs=_bq((QP, D)),
        out_shape=jax.ShapeDtypeStruct((B, QP, D), f32),
        scratch_shapes=[pltpu.VMEM((QP, D), f32)],
    )(keys, tt, mm, den, memory, qpad,
      p['gate_w1'], p['gate_b1'], p['gate_w2'], p['gate_b2'])

    qln, cq = pl.pallas_call(
        _k3_body,
        grid=(B,),
        in_specs=[_bq((QP, D)), _bq((QP, D)),
                  _full((3 * D, D)), _full((3 * D,)),
                  _full((D, D)), _full((D,)),
                  _full((D,)), _full((D,)), _full((D, D))],
        out_specs=[_bq((QP, D)), _bq((QP, D))],
        out_shape=[jax.ShapeDtypeStruct((B, QP, D), f32),
                   jax.ShapeDtypeStruct((B, QP, D), f32)],
    )(q1, qpos, p['sa_wi'], p['sa_bi'], p['sa_wo'], p['sa_bo'],
      p['ln1_w'], p['ln1_b'], p['cross_wq'])

    imp = pl.pallas_call(
        _k4a_body,
        grid=(B, NT),
        in_specs=[_bq((QP, D)),
                  pl.BlockSpec((1, TN, D), lambda b, n: (b, n, 0))],
        out_specs=pl.BlockSpec((1, 1, TN), lambda b, n: (b, 0, n)),
        out_shape=jax.ShapeDtypeStruct((B, 1, N), f32),
    )(cq, cm)

    bias = pl.pallas_call(
        _k4b_body,
        grid=(B,),
        in_specs=[_bq((1, N))],
        out_specs=_bq((1, N)),
        out_shape=jax.ShapeDtypeStruct((B, 1, N), f32),
    )(imp)

    out = pl.pallas_call(
        _k5_body,
        grid=(B, NT),
        in_specs=[_bq((QP, D)), _bq((QP, D)),
                  pl.BlockSpec((1, TN, D), lambda b, n: (b, n, 0)),
                  pl.BlockSpec((1, 1, TN), lambda b, n: (b, 0, n)),
                  _full((3 * D, D)), _full((3 * D,)),
                  _full((D, D)), _full((D,)),
                  _full((D,)), _full((D,)),
                  _full((4 * D, D)), _full((4 * D,)),
                  _full((D, 4 * D)), _full((D,)),
                  _full((D,)), _full((D,))],
        out_specs=_bq((QP, D)),
        out_shape=jax.ShapeDtypeStruct((B, QP, D), f32),
        scratch_shapes=[pltpu.VMEM((QP, D), f32),
                        pltpu.VMEM((QP, NHEAD), f32),
                        pltpu.VMEM((QP, NHEAD), f32)],
    )(qln, qpos, memory, bias,
      p['ca_wi'], p['ca_bi'], p['ca_wo'], p['ca_bo'],
      p['ln2_w'], p['ln2_b'],
      p['ffn_w1'], p['ffn_b1'], p['ffn_w2'], p['ffn_b2'],
      p['ln3_w'], p['ln3_b'])
    return out[:, :Q, :]


# speedup vs baseline: 1.0402x; 1.0402x over previous
import math

import jax
import jax.numpy as jnp
from jax.experimental import pallas as pl
from jax.experimental.pallas import tpu as pltpu

B, Q, N, D = 4, 300, 16384, 256
QP = 304
NHEAD = 8
DH = D // NHEAD
ALIGN_TOPK = 96
CROSS_TOPK = 2048
NEG = -1e30
TN = 2048
NT = N // TN
TR = 64
SCALE = 1.0 / math.sqrt(DH)


def _f32_to_key(x):
    u = jax.lax.bitcast_convert_type(x, jnp.int32)
    return jnp.where(u >= 0, u, u ^ jnp.int32(0x7FFFFFFF))


def _key_to_f32(k):
    u = jnp.where(k >= 0, k, k ^ jnp.int32(0x7FFFFFFF))
    return jax.lax.bitcast_convert_type(u, jnp.float32)


def _kth_largest_key(keys, k):
    def count_ge(t):
        return jnp.sum((keys >= t).astype(jnp.int32), axis=-1, keepdims=True)

    t = jnp.where(count_ge(jnp.int32(0)) >= k,
                  jnp.int32(0), jnp.int32(-2147483648))
    t = jnp.where(t < 0, t | jnp.int32(1 << 30), t)
    for b in range(29, -1, -1):
        trial = t | jnp.int32(1 << b)
        t = jnp.where(count_ge(trial) >= k, trial, t)
    return t


def _layer_norm(x, w, b, eps=1e-5):
    mu = jnp.mean(x, axis=-1, keepdims=True)
    xc = x - mu
    var = jnp.mean(xc * xc, axis=-1, keepdims=True)
    return xc * jax.lax.rsqrt(var + eps) * w + b


def _gelu(x):
    return 0.5 * x * (1.0 + jax.lax.erf(x * (1.0 / math.sqrt(2.0))))


def _normalize(x, eps=1e-6):
    n = jnp.sqrt(jnp.sum(x * x, axis=-1, keepdims=True))
    return x / jnp.maximum(n, eps)


def _softmax(x):
    m = jnp.max(x, axis=-1, keepdims=True)
    e = jnp.exp(x - m)
    return e / jnp.sum(e, axis=-1, keepdims=True)


def _k1_body(q_ref, awq_ref, mem_ref, awm_ref, cwm_ref, keys_ref, cm_ref,
             qn_ref):
    nt = pl.program_id(1)

    @pl.when(nt == 0)
    def _():
        qn_ref[...] = _normalize(q_ref[0] @ awq_ref[...].T)

    mem = mem_ref[0]
    mn = _normalize(mem @ awm_ref[...].T)
    cm_ref[0] = _normalize(mem @ cwm_ref[...].T)
    keys_ref[0] = _f32_to_key(qn_ref[...] @ mn.T)


def _k2a_body(keys_ref, t_ref, m_ref, den_ref):
    keys = keys_ref[...]
    t = _kth_largest_key(keys, ALIGN_TOPK)
    m_key = jnp.max(keys, axis=-1, keepdims=True)
    m = _key_to_f32(m_key)
    sim = _key_to_f32(keys)
    den = jnp.sum(jnp.where(keys >= t, jnp.exp(sim - m), 0.0),
                  axis=-1, keepdims=True)
    t_ref[...] = t
    m_ref[...] = m
    den_ref[...] = den


def _k2b_body(keys_ref, t_ref, m_ref, den_ref, mem_ref, q_ref,
              gw1_ref, gb1_ref, gw2_ref, gb2_ref, out_ref, acc_ref):
    nt = pl.program_id(1)
    keys = keys_ref[0]
    sim = _key_to_f32(keys)
    w = jnp.where(keys >= t_ref[0], jnp.exp(sim - m_ref[0]), 0.0) / den_ref[0]
    part = w @ mem_ref[0]

    @pl.when(nt == 0)
    def _():
        acc_ref[...] = jnp.zeros_like(acc_ref)

    acc_ref[...] += part

    @pl.when(nt == NT - 1)
    def _():
        aligned = acc_ref[...]
        qb = q_ref[0]
        h = (jnp.concatenate([qb, aligned], axis=-1) @ gw1_ref[...].T
             + gb1_ref[...])
        gate = jax.nn.sigmoid(_gelu(h) @ gw2_ref[...].T + gb2_ref[...])
        out_ref[0] = qb + gate * aligned


def _k3_body(q_ref, qpos_ref, wi_ref, bi_ref, wo_ref, bo_ref,
             ln1w_ref, ln1b_ref, cwq_ref, qln_ref, cq_ref):
    qb = q_ref[0]
    qk = qb + qpos_ref[0]
    wi = wi_ref[...]
    bi = bi_ref[...]
    qh = qk @ wi[:D].T + bi[:D]
    kh = qk @ wi[D:2 * D].T + bi[D:2 * D]
    vh = qb @ wi[2 * D:].T + bi[2 * D:]
    colmask = jnp.where(
        jax.lax.broadcasted_iota(jnp.int32, (QP, QP), 1) < Q, 0.0, NEG)
    outs = []
    for h in range(NHEAD):
        s = slice(h * DH, (h + 1) * DH)
        sc = (qh[:, s] @ kh[:, s].T) * SCALE + colmask
        outs.append(_softmax(sc) @ vh[:, s])
    o = jnp.concatenate(outs, axis=-1) @ wo_ref[...].T + bo_ref[...]
    qln = _layer_norm(qb + o, ln1w_ref[...], ln1b_ref[...])
    qln_ref[0] = qln
    cq_ref[0] = _normalize(qln @ cwq_ref[...].T)


def _k4_body(cq_ref, cm_ref, bias_ref, imp_ref):
    nt = pl.program_id(1)
    sim2 = cq_ref[0] @ cm_ref[0].T
    rowmask = jnp.where(
        jax.lax.broadcasted_iota(jnp.int32, (QP, TN), 0) < Q, 0.0, NEG)
    imp_ref[:, pl.ds(nt * TN, TN)] = jnp.max(
        sim2 + rowmask, axis=0, keepdims=True)

    @pl.when(nt == NT - 1)
    def _():
        keys = _f32_to_key(imp_ref[...])
        t = _kth_largest_key(keys, CROSS_TOPK)
        bias_ref[0] = jnp.where(keys >= t, 0.0, NEG)


def _k5_body(qln_ref, qpos_ref, mem_ref, bias_ref,
             wi_ref, bi_ref, wo_ref, bo_ref,
             ln2w_ref, ln2b_ref, fw1_ref, fb1_ref, fw2_ref, fb2_ref,
             ln3w_ref, ln3b_ref, out_ref, acc_ref, m_ref, l_ref):
    nt = pl.program_id(1)
    qln = qln_ref[0]
    qi = qln + qpos_ref[0]
    mem = mem_ref[0]
    bias = bias_ref[0]
    wi = wi_ref[...]
    bi = bi_ref[...]

    @pl.when(nt == 0)
    def _():
        acc_ref[...] = jnp.zeros_like(acc_ref)
        m_ref[...] = jnp.full_like(m_ref, NEG)
        l_ref[...] = jnp.zeros_like(l_ref)

    for h in range(NHEAD):
        s = slice(h * DH, (h + 1) * DH)
        qh = qi @ wi[:D][s].T + bi[:D][s]
        kh = mem @ wi[D:2 * D][s].T + bi[D:2 * D][s]
        vh = mem @ wi[2 * D:][s].T + bi[2 * D:][s]
        sc = (qh @ kh.T) * SCALE + bias
        m_old = m_ref[:, h:h + 1]
        m_new = jnp.maximum(m_old, jnp.max(sc, axis=-1, keepdims=True))
        p = jnp.exp(sc - m_new)
        corr = jnp.exp(m_old - m_new)
        l_ref[:, h:h + 1] = l_ref[:, h:h + 1] * corr + jnp.sum(
            p, axis=-1, keepdims=True)
        acc_ref[:, s] = acc_ref[:, s] * corr + p @ vh
        m_ref[:, h:h + 1] = m_new

    @pl.when(nt == NT - 1)
    def _():
        outs = []
        for h in range(NHEAD):
            s = slice(h * DH, (h + 1) * DH)
            outs.append(acc_ref[:, s] / l_ref[:, h:h + 1])
        o = jnp.concatenate(outs, axis=-1) @ wo_ref[...].T + bo_ref[...]
        x = _layer_norm(qln + o, ln2w_ref[...], ln2b_ref[...])
        ff = (_gelu(x @ fw1_ref[...].T + fb1_ref[...]) @ fw2_ref[...].T
              + fb2_ref[...])
        out_ref[0] = _layer_norm(x + ff, ln3w_ref[...], ln3b_ref[...])


def _full(arr_shape):
    return pl.BlockSpec(arr_shape, lambda *_: (0,) * len(arr_shape))


def _bq(shape):
    return pl.BlockSpec((1,) + shape, lambda b, *_: (b, 0, 0))


@jax.jit
def kernel(q, q_pos, memory, params):
    p = params
    f32 = jnp.float32
    qpad = jnp.pad(q, ((0, 0), (0, QP - Q), (0, 0)))
    qpos = jnp.pad(q_pos, ((0, 0), (0, QP - Q), (0, 0)))

    keys, cm = pl.pallas_call(
        _k1_body,
        grid=(B, NT),
        in_specs=[_bq((QP, D)), _full((D, D)),
                  pl.BlockSpec((1, TN, D), lambda b, n: (b, n, 0)),
                  _full((D, D)), _full((D, D))],
        out_specs=[pl.BlockSpec((1, QP, TN), lambda b, n: (b, 0, n)),
                   pl.BlockSpec((1, TN, D), lambda b, n: (b, n, 0))],
        out_shape=[jax.ShapeDtypeStruct((B, QP, N), jnp.int32),
                   jax.ShapeDtypeStruct((B, N, D), f32)],
        scratch_shapes=[pltpu.VMEM((QP, D), f32)],
    )(qpad, p['align_wq'], memory, p['align_wm'], p['cross_wm'])

    keys2d = keys.reshape(B * QP, N)
    RT = (B * QP) // TR
    tt, mm, den = pl.pallas_call(
        _k2a_body,
        grid=(RT,),
        in_specs=[pl.BlockSpec((TR, N), lambda r: (r, 0))],
        out_specs=[pl.BlockSpec((TR, 1), lambda r: (r, 0))] * 3,
        out_shape=[jax.ShapeDtypeStruct((B * QP, 1), jnp.int32),
                   jax.ShapeDtypeStruct((B * QP, 1), f32),
                   jax.ShapeDtypeStruct((B * QP, 1), f32)],
    )(keys2d)
    tt = tt.reshape(B, QP, 1)
    mm = mm.reshape(B, QP, 1)
    den = den.reshape(B, QP, 1)

    q1 = pl.pallas_call(
        _k2b_body,
        grid=(B, NT),
        in_specs=[pl.BlockSpec((1, QP, TN), lambda b, n: (b, 0, n)),
                  _bq((QP, 1)), _bq((QP, 1)), _bq((QP, 1)),
                  pl.BlockSpec((1, TN, D), lambda b, n: (b, n, 0)),
                  _bq((QP, D)),
                  _full((D, 2 * D)), _full((D,)), _full((D, D)), _full((D,))],
        out_specs=_bq((QP, D)),
        out_shape=jax.ShapeDtypeStruct((B, QP, D), f32),
        scratch_shapes=[pltpu.VMEM((QP, D), f32)],
    )(keys, tt, mm, den, memory, qpad,
      p['gate_w1'], p['gate_b1'], p['gate_w2'], p['gate_b2'])

    qln, cq = pl.pallas_call(
        _k3_body,
        grid=(B,),
        in_specs=[_bq((QP, D)), _bq((QP, D)),
                  _full((3 * D, D)), _full((3 * D,)),
                  _full((D, D)), _full((D,)),
                  _full((D,)), _full((D,)), _full((D, D))],
        out_specs=[_bq((QP, D)), _bq((QP, D))],
        out_shape=[jax.ShapeDtypeStruct((B, QP, D), f32),
                   jax.ShapeDtypeStruct((B, QP, D), f32)],
    )(q1, qpos, p['sa_wi'], p['sa_bi'], p['sa_wo'], p['sa_bo'],
      p['ln1_w'], p['ln1_b'], p['cross_wq'])

    bias = pl.pallas_call(
        _k4_body,
        grid=(B, NT),
        in_specs=[_bq((QP, D)),
                  pl.BlockSpec((1, TN, D), lambda b, n: (b, n, 0))],
        out_specs=pl.BlockSpec((1, 1, N), lambda b, n: (b, 0, 0)),
        out_shape=jax.ShapeDtypeStruct((B, 1, N), f32),
        scratch_shapes=[pltpu.VMEM((1, N), f32)],
    )(cq, cm)

    out = pl.pallas_call(
        _k5_body,
        grid=(B, NT),
        in_specs=[_bq((QP, D)), _bq((QP, D)),
                  pl.BlockSpec((1, TN, D), lambda b, n: (b, n, 0)),
                  pl.BlockSpec((1, 1, TN), lambda b, n: (b, 0, n)),
                  _full((3 * D, D)), _full((3 * D,)),
                  _full((D, D)), _full((D,)),
                  _full((D,)), _full((D,)),
                  _full((4 * D, D)), _full((4 * D,)),
                  _full((D, 4 * D)), _full((D,)),
                  _full((D,)), _full((D,))],
        out_specs=_bq((QP, D)),
        out_shape=jax.ShapeDtypeStruct((B, QP, D), f32),
        scratch_shapes=[pltpu.VMEM((QP, D), f32),
                        pltpu.VMEM((QP, NHEAD), f32),
                        pltpu.VMEM((QP, NHEAD), f32)],
    )(qln, qpos, memory, bias,
      p['ca_wi'], p['ca_bi'], p['ca_wo'], p['ca_bo'],
      p['ln2_w'], p['ln2_b'],
      p['ffn_w1'], p['ffn_b1'], p['ffn_w2'], p['ffn_b2'],
      p['ln3_w'], p['ln3_b'])
    return out[:, :Q, :]


# speedup vs baseline: 1.1018x; 1.0592x over previous
import math

import jax
import jax.numpy as jnp
from jax.experimental import pallas as pl
from jax.experimental.pallas import tpu as pltpu

B, Q, N, D = 4, 300, 16384, 256
QP = 304
NHEAD = 8
DH = D // NHEAD
ALIGN_TOPK = 96
CROSS_TOPK = 2048
NEG = -1e30
TN = 4096
NT = N // TN
TR = 64
SCALE = 1.0 / math.sqrt(DH)


def _f32_to_key(x):
    u = jax.lax.bitcast_convert_type(x, jnp.int32)
    return jnp.where(u >= 0, u, u ^ jnp.int32(0x7FFFFFFF))


def _key_to_f32(k):
    u = jnp.where(k >= 0, k, k ^ jnp.int32(0x7FFFFFFF))
    return jax.lax.bitcast_convert_type(u, jnp.float32)


def _kth_largest_key(keys, k):
    def count_ge(t):
        return jnp.sum((keys >= t).astype(jnp.int32), axis=-1, keepdims=True)

    t = jnp.where(count_ge(jnp.int32(0)) >= k,
                  jnp.int32(0), jnp.int32(-2147483648))
    t = jnp.where(t < 0, t | jnp.int32(1 << 30), t)
    for b in range(29, -1, -1):
        trial = t | jnp.int32(1 << b)
        t = jnp.where(count_ge(trial) >= k, trial, t)
    return t


def _layer_norm(x, w, b, eps=1e-5):
    mu = jnp.mean(x, axis=-1, keepdims=True)
    xc = x - mu
    var = jnp.mean(xc * xc, axis=-1, keepdims=True)
    return xc * jax.lax.rsqrt(var + eps) * w + b


def _gelu(x):
    return 0.5 * x * (1.0 + jax.lax.erf(x * (1.0 / math.sqrt(2.0))))


def _normalize(x, eps=1e-6):
    n = jnp.sqrt(jnp.sum(x * x, axis=-1, keepdims=True))
    return x / jnp.maximum(n, eps)


def _softmax(x):
    m = jnp.max(x, axis=-1, keepdims=True)
    e = jnp.exp(x - m)
    return e / jnp.sum(e, axis=-1, keepdims=True)


def _k1_body(q_ref, awq_ref, mem_ref, awm_ref, cwm_ref, keys_ref, cm_ref,
             qn_ref):
    nt = pl.program_id(1)

    @pl.when(nt == 0)
    def _():
        qn_ref[...] = _normalize(q_ref[0] @ awq_ref[...].T)

    mem = mem_ref[0]
    mn = _normalize(mem @ awm_ref[...].T)
    cm_ref[0] = _normalize(mem @ cwm_ref[...].T)
    keys_ref[0] = _f32_to_key(qn_ref[...] @ mn.T)


def _k2a_body(keys_ref, t_ref, m_ref, den_ref):
    keys = keys_ref[...]
    t = _kth_largest_key(keys, ALIGN_TOPK)
    m_key = jnp.max(keys, axis=-1, keepdims=True)
    m = _key_to_f32(m_key)
    sim = _key_to_f32(keys)
    den = jnp.sum(jnp.where(keys >= t, jnp.exp(sim - m), 0.0),
                  axis=-1, keepdims=True)
    t_ref[...] = t
    m_ref[...] = m
    den_ref[...] = den


def _k2b_body(keys_ref, t_ref, m_ref, den_ref, mem_ref, q_ref,
              gw1_ref, gb1_ref, gw2_ref, gb2_ref, out_ref, acc_ref):
    nt = pl.program_id(1)
    keys = keys_ref[0]
    sim = _key_to_f32(keys)
    w = jnp.where(keys >= t_ref[0], jnp.exp(sim - m_ref[0]), 0.0) / den_ref[0]
    part = w @ mem_ref[0]

    @pl.when(nt == 0)
    def _():
        acc_ref[...] = jnp.zeros_like(acc_ref)

    acc_ref[...] += part

    @pl.when(nt == NT - 1)
    def _():
        aligned = acc_ref[...]
        qb = q_ref[0]
        h = (jnp.concatenate([qb, aligned], axis=-1) @ gw1_ref[...].T
             + gb1_ref[...])
        gate = jax.nn.sigmoid(_gelu(h) @ gw2_ref[...].T + gb2_ref[...])
        out_ref[0] = qb + gate * aligned


def _k3_body(q_ref, qpos_ref, wi_ref, bi_ref, wo_ref, bo_ref,
             ln1w_ref, ln1b_ref, cwq_ref, qln_ref, cq_ref):
    qb = q_ref[0]
    qk = qb + qpos_ref[0]
    wi = wi_ref[...]
    bi = bi_ref[...]
    qh = qk @ wi[:D].T + bi[:D]
    kh = qk @ wi[D:2 * D].T + bi[D:2 * D]
    vh = qb @ wi[2 * D:].T + bi[2 * D:]
    colmask = jnp.where(
        jax.lax.broadcasted_iota(jnp.int32, (QP, QP), 1) < Q, 0.0, NEG)
    outs = []
    for h in range(NHEAD):
        s = slice(h * DH, (h + 1) * DH)
        sc = (qh[:, s] @ kh[:, s].T) * SCALE + colmask
        outs.append(_softmax(sc) @ vh[:, s])
    o = jnp.concatenate(outs, axis=-1) @ wo_ref[...].T + bo_ref[...]
    qln = _layer_norm(qb + o, ln1w_ref[...], ln1b_ref[...])
    qln_ref[0] = qln
    cq_ref[0] = _normalize(qln @ cwq_ref[...].T)


def _k4_body(cq_ref, cm_ref, bias_ref, imp_ref):
    nt = pl.program_id(1)
    sim2 = cq_ref[0] @ cm_ref[0].T
    rowmask = jnp.where(
        jax.lax.broadcasted_iota(jnp.int32, (QP, TN), 0) < Q, 0.0, NEG)
    imp_ref[:, pl.ds(nt * TN, TN)] = jnp.max(
        sim2 + rowmask, axis=0, keepdims=True)

    @pl.when(nt == NT - 1)
    def _():
        keys = _f32_to_key(imp_ref[...])
        t = _kth_largest_key(keys, CROSS_TOPK)
        bias_ref[0] = jnp.where(keys >= t, 0.0, NEG)


def _k5_body(qln_ref, qpos_ref, mem_ref, bias_ref,
             wi_ref, bi_ref, wo_ref, bo_ref,
             ln2w_ref, ln2b_ref, fw1_ref, fb1_ref, fw2_ref, fb2_ref,
             ln3w_ref, ln3b_ref, out_ref, acc_ref, m_ref, l_ref):
    nt = pl.program_id(1)
    qln = qln_ref[0]
    qi = qln + qpos_ref[0]
    mem = mem_ref[0]
    bias = bias_ref[0]
    wi = wi_ref[...]
    bi = bi_ref[...]

    @pl.when(nt == 0)
    def _():
        acc_ref[...] = jnp.zeros_like(acc_ref)
        m_ref[...] = jnp.full_like(m_ref, NEG)
        l_ref[...] = jnp.zeros_like(l_ref)

    for h in range(NHEAD):
        s = slice(h * DH, (h + 1) * DH)
        qh = qi @ wi[:D][s].T + bi[:D][s]
        kh = mem @ wi[D:2 * D][s].T + bi[D:2 * D][s]
        vh = mem @ wi[2 * D:][s].T + bi[2 * D:][s]
        sc = (qh @ kh.T) * SCALE + bias
        m_old = m_ref[:, h:h + 1]
        m_new = jnp.maximum(m_old, jnp.max(sc, axis=-1, keepdims=True))
        p = jnp.exp(sc - m_new)
        corr = jnp.exp(m_old - m_new)
        l_ref[:, h:h + 1] = l_ref[:, h:h + 1] * corr + jnp.sum(
            p, axis=-1, keepdims=True)
        acc_ref[:, s] = acc_ref[:, s] * corr + p @ vh
        m_ref[:, h:h + 1] = m_new

    @pl.when(nt == NT - 1)
    def _():
        outs = []
        for h in range(NHEAD):
            s = slice(h * DH, (h + 1) * DH)
            outs.append(acc_ref[:, s] / l_ref[:, h:h + 1])
        o = jnp.concatenate(outs, axis=-1) @ wo_ref[...].T + bo_ref[...]
        x = _layer_norm(qln + o, ln2w_ref[...], ln2b_ref[...])
        ff = (_gelu(x @ fw1_ref[...].T + fb1_ref[...]) @ fw2_ref[...].T
              + fb2_ref[...])
        out_ref[0] = _layer_norm(x + ff, ln3w_ref[...], ln3b_ref[...])


def _full(arr_shape):
    return pl.BlockSpec(arr_shape, lambda *_: (0,) * len(arr_shape))


def _bq(shape):
    return pl.BlockSpec((1,) + shape, lambda b, *_: (b, 0, 0))


@jax.jit
def kernel(q, q_pos, memory, params):
    p = params
    f32 = jnp.float32
    qpad = jnp.pad(q, ((0, 0), (0, QP - Q), (0, 0)))
    qpos = jnp.pad(q_pos, ((0, 0), (0, QP - Q), (0, 0)))

    keys, cm = pl.pallas_call(
        _k1_body,
        grid=(B, NT),
        in_specs=[_bq((QP, D)), _full((D, D)),
                  pl.BlockSpec((1, TN, D), lambda b, n: (b, n, 0)),
                  _full((D, D)), _full((D, D))],
        out_specs=[pl.BlockSpec((1, QP, TN), lambda b, n: (b, 0, n)),
                   pl.BlockSpec((1, TN, D), lambda b, n: (b, n, 0))],
        out_shape=[jax.ShapeDtypeStruct((B, QP, N), jnp.int32),
                   jax.ShapeDtypeStruct((B, N, D), f32)],
        scratch_shapes=[pltpu.VMEM((QP, D), f32)],
    )(qpad, p['align_wq'], memory, p['align_wm'], p['cross_wm'])

    keys2d = keys.reshape(B * QP, N)
    RT = (B * QP) // TR
    tt, mm, den = pl.pallas_call(
        _k2a_body,
        grid=(RT,),
        in_specs=[pl.BlockSpec((TR, N), lambda r: (r, 0))],
        out_specs=[pl.BlockSpec((TR, 1), lambda r: (r, 0))] * 3,
        out_shape=[jax.ShapeDtypeStruct((B * QP, 1), jnp.int32),
                   jax.ShapeDtypeStruct((B * QP, 1), f32),
                   jax.ShapeDtypeStruct((B * QP, 1), f32)],
    )(keys2d)
    tt = tt.reshape(B, QP, 1)
    mm = mm.reshape(B, QP, 1)
    den = den.reshape(B, QP, 1)

    q1 = pl.pallas_call(
        _k2b_body,
        grid=(B, NT),
        in_specs=[pl.BlockSpec((1, QP, TN), lambda b, n: (b, 0, n)),
                  _bq((QP, 1)), _bq((QP, 1)), _bq((QP, 1)),
                  pl.BlockSpec((1, TN, D), lambda b, n: (b, n, 0)),
                  _bq((QP, D)),
                  _full((D, 2 * D)), _full((D,)), _full((D, D)), _full((D,))],
        out_specs=_bq((QP, D)),
        out_shape=jax.ShapeDtypeStruct((B, QP, D), f32),
        scratch_shapes=[pltpu.VMEM((QP, D), f32)],
    )(keys, tt, mm, den, memory, qpad,
      p['gate_w1'], p['gate_b1'], p['gate_w2'], p['gate_b2'])

    qln, cq = pl.pallas_call(
        _k3_body,
        grid=(B,),
        in_specs=[_bq((QP, D)), _bq((QP, D)),
                  _full((3 * D, D)), _full((3 * D,)),
                  _full((D, D)), _full((D,)),
                  _full((D,)), _full((D,)), _full((D, D))],
        out_specs=[_bq((QP, D)), _bq((QP, D))],
        out_shape=[jax.ShapeDtypeStruct((B, QP, D), f32),
                   jax.ShapeDtypeStruct((B, QP, D), f32)],
    )(q1, qpos, p['sa_wi'], p['sa_bi'], p['sa_wo'], p['sa_bo'],
      p['ln1_w'], p['ln1_b'], p['cross_wq'])

    bias = pl.pallas_call(
        _k4_body,
        grid=(B, NT),
        in_specs=[_bq((QP, D)),
                  pl.BlockSpec((1, TN, D), lambda b, n: (b, n, 0))],
        out_specs=pl.BlockSpec((1, 1, N), lambda b, n: (b, 0, 0)),
        out_shape=jax.ShapeDtypeStruct((B, 1, N), f32),
        scratch_shapes=[pltpu.VMEM((1, N), f32)],
    )(cq, cm)

    out = pl.pallas_call(
        _k5_body,
        grid=(B, NT),
        in_specs=[_bq((QP, D)), _bq((QP, D)),
                  pl.BlockSpec((1, TN, D), lambda b, n: (b, n, 0)),
                  pl.BlockSpec((1, 1, TN), lambda b, n: (b, 0, n)),
                  _full((3 * D, D)), _full((3 * D,)),
                  _full((D, D)), _full((D,)),
                  _full((D,)), _full((D,)),
                  _full((4 * D, D)), _full((4 * D,)),
                  _full((D, 4 * D)), _full((D,)),
                  _full((D,)), _full((D,))],
        out_specs=_bq((QP, D)),
        out_shape=jax.ShapeDtypeStruct((B, QP, D), f32),
        scratch_shapes=[pltpu.VMEM((QP, D), f32),
                        pltpu.VMEM((QP, NHEAD), f32),
                        pltpu.VMEM((QP, NHEAD), f32)],
    )(qln, qpos, memory, bias,
      p['ca_wi'], p['ca_bi'], p['ca_wo'], p['ca_bo'],
      p['ln2_w'], p['ln2_b'],
      p['ffn_w1'], p['ffn_b1'], p['ffn_w2'], p['ffn_b2'],
      p['ln3_w'], p['ln3_b'])
    return out[:, :Q, :]
